# Initial kernel scaffold; baseline (speedup 1.0000x reference)
import jax
import jax.numpy as jnp
from jax.experimental import pallas as pl


def kernel(x, edge_index, batch, W1_l, b1_l, W1_r, W2_l, b2_l, W2_r):
    raise NotImplementedError("write your pallas kernel here")


# speedup vs baseline: 1.0375x; 1.0375x over previous
import functools

import jax
import jax.numpy as jnp
from jax.experimental import pallas as pl
from jax.experimental.pallas import tpu as pltpu

N = 10000
E = 320000
D_IN = 128
D_H = 256
D_OUT = 256
G = 64

N_PAD = 10240
R = 1024
NB = N_PAD // R


def _dense1_body(x_ref, agg_ref, cnt_ref, w1lt_ref, b1_ref, w1rt_ref,
                 h1a_ref, h1b_ref):
    inv = 1.0 / jnp.maximum(cnt_ref[...], 1.0)
    mean = agg_ref[...] * inv
    h = jnp.dot(mean, w1lt_ref[...], preferred_element_type=jnp.float32)
    h += jnp.dot(x_ref[...], w1rt_ref[...], preferred_element_type=jnp.float32)
    h += b1_ref[...]
    h = jnp.maximum(h, 0.0)
    h1a_ref[...] = h[:, :D_H // 2]
    h1b_ref[...] = h[:, D_H // 2:]


def _dense1(x_pad, agg1_pad, cnt_col, w1lt, b1_row, w1rt):
    return pl.pallas_call(
        _dense1_body,
        grid=(NB,),
        in_specs=[
            pl.BlockSpec((R, D_IN), lambda i: (i, 0)),
            pl.BlockSpec((R, D_IN), lambda i: (i, 0)),
            pl.BlockSpec((R, 1), lambda i: (i, 0)),
            pl.BlockSpec((D_IN, D_H), lambda i: (0, 0)),
            pl.BlockSpec((1, D_H), lambda i: (0, 0)),
            pl.BlockSpec((D_IN, D_H), lambda i: (0, 0)),
        ],
        out_specs=[
            pl.BlockSpec((R, D_H // 2), lambda i: (i, 0)),
            pl.BlockSpec((R, D_H // 2), lambda i: (i, 0)),
        ],
        out_shape=[
            jax.ShapeDtypeStruct((N_PAD, D_H // 2), jnp.float32),
            jax.ShapeDtypeStruct((N_PAD, D_H // 2), jnp.float32),
        ],
    )(x_pad, agg1_pad, cnt_col, w1lt, b1_row, w1rt)


def _dense2_body(h1a_ref, h1b_ref, a2a_ref, a2b_ref, batch_ref, cntr_ref,
                 w2lt_ref, b2_ref, w2rt_ref, out_ref,
                 acc_m2, acc_h1, acc_g):
    j = pl.program_id(0)

    @pl.when(j == 0)
    def _():
        acc_m2[...] = jnp.zeros_like(acc_m2)
        acc_h1[...] = jnp.zeros_like(acc_h1)
        acc_g[...] = jnp.zeros_like(acc_g)

    batch_row = batch_ref[0]
    mask = (jax.lax.broadcasted_iota(jnp.int32, (G, R), 0)
            == batch_row).astype(jnp.float32)
    invr = 1.0 / jnp.maximum(cntr_ref[0], 1.0)
    maskw = mask * invr

    h1 = jnp.concatenate([h1a_ref[...], h1b_ref[...]], axis=1)
    agg2 = jnp.concatenate([a2a_ref[...], a2b_ref[...]], axis=1)

    acc_m2[...] += jnp.dot(maskw, agg2, preferred_element_type=jnp.float32)
    acc_h1[...] += jnp.dot(mask, h1, preferred_element_type=jnp.float32)
    acc_g[...] += jnp.sum(mask, axis=1, keepdims=True)

    @pl.when(j == pl.num_programs(0) - 1)
    def _():
        out = jnp.dot(acc_m2[...], w2lt_ref[...],
                      preferred_element_type=jnp.float32)
        out += jnp.dot(acc_h1[...], w2rt_ref[...],
                       preferred_element_type=jnp.float32)
        out += acc_g[...] * b2_ref[...]
        out_ref[...] = out


def _dense2(h1a, h1b, a2a, a2b, batch3d, cnt3d, w2lt, b2_row, w2rt):
    return pl.pallas_call(
        _dense2_body,
        grid=(NB,),
        in_specs=[
            pl.BlockSpec((R, D_H // 2), lambda i: (i, 0)),
            pl.BlockSpec((R, D_H // 2), lambda i: (i, 0)),
            pl.BlockSpec((R, D_H // 2), lambda i: (i, 0)),
            pl.BlockSpec((R, D_H // 2), lambda i: (i, 0)),
            pl.BlockSpec((1, 1, R), lambda i: (i, 0, 0)),
            pl.BlockSpec((1, 1, R), lambda i: (i, 0, 0)),
            pl.BlockSpec((D_H, D_OUT), lambda i: (0, 0)),
            pl.BlockSpec((1, D_OUT), lambda i: (0, 0)),
            pl.BlockSpec((D_H, D_OUT), lambda i: (0, 0)),
        ],
        out_specs=pl.BlockSpec((G, D_OUT), lambda i: (0, 0)),
        out_shape=jax.ShapeDtypeStruct((G, D_OUT), jnp.float32),
        scratch_shapes=[
            pltpu.VMEM((G, D_OUT), jnp.float32),
            pltpu.VMEM((G, D_OUT), jnp.float32),
            pltpu.VMEM((G, 1), jnp.float32),
        ],
    )(h1a, h1b, a2a, a2b, batch3d, cnt3d, w2lt, b2_row, w2rt)


@jax.jit
def kernel(x, edge_index, batch, W1_l, b1_l, W1_r, W2_l, b2_l, W2_r):
    src = edge_index[0]
    dst = edge_index[1]

    cnt = jax.ops.segment_sum(jnp.ones((E,), jnp.float32), dst, num_segments=N)
    agg1 = jax.ops.segment_sum(jnp.take(x, src, axis=0), dst, num_segments=N)

    pad = N_PAD - N
    x_pad = jnp.pad(x, ((0, pad), (0, 0)))
    agg1_pad = jnp.pad(agg1, ((0, pad), (0, 0)))
    cnt_pad = jnp.pad(cnt, (0, pad))
    cnt_col = cnt_pad[:, None]

    h1a, h1b = _dense1(x_pad, agg1_pad, cnt_col,
                       W1_l.T, b1_l[None, :], W1_r.T)

    h1 = jnp.concatenate([h1a[:N], h1b[:N]], axis=1)
    agg2 = jax.ops.segment_sum(jnp.take(h1, src, axis=0), dst, num_segments=N)
    agg2_pad = jnp.pad(agg2, ((0, pad), (0, 0)))
    a2a = agg2_pad[:, :D_H // 2]
    a2b = agg2_pad[:, D_H // 2:]

    batch_pad = jnp.pad(batch, (0, pad), constant_values=G)
    batch3d = batch_pad.reshape(NB, 1, R)
    cnt3d = cnt_pad.reshape(NB, 1, R)

    return _dense2(h1a, h1b, a2a, a2b, batch3d, cnt3d,
                   W2_l.T, b2_l[None, :], W2_r.T)


# speedup vs baseline: 5.5278x; 5.3281x over previous
import dataclasses
import functools

import jax
import jax.numpy as jnp
from jax import lax
from jax.experimental import pallas as pl
from jax.experimental.pallas import tpu as pltpu
from jax.experimental.pallas import tpu_sc as plsc

N = 10000
E = 320000
D_IN = 128
D_H = 256
D_OUT = 256
G = 64

R = 1000
NB = N // R

NSUB = 16
NTILE = 2 * NSUB
W = 80
N_ACC = 10240
RPT = N_ACC // NSUB
ZR = 32

_MESH = plsc.VectorSubcoreMesh(core_axis_name="c", subcore_axis_name="s")

_SC_CP = pltpu.CompilerParams()
if "needs_layout_passes" in pltpu.CompilerParams.__dataclass_fields__:
    _SC_CP = dataclasses.replace(_SC_CP, needs_layout_passes=False)


def _sc_agg_body(edge_split, *refs):
    (tab, src_hbm, dst_hbm, agg_a, agg_b,
     is0, id0, is1, id1, rows0, rows1, zbuf, acc,
     gsem0, gsem1) = refs

    c = lax.axis_index("c")
    s = lax.axis_index("s")
    row0 = s * RPT
    d_half = rows0.shape[1]

    @pl.loop(0, ZR)
    def _(r):
        @pl.loop(0, d_half, step=16)
        def _(cc):
            zbuf[r, pl.ds(cc, 16)] = jnp.zeros((16,), jnp.float32)

    @pl.loop(0, RPT // ZR)
    def _(i):
        pltpu.sync_copy(zbuf, acc.at[pl.ds(row0 + i * ZR, ZR)])

    plsc.subcore_barrier()

    if edge_split:
        ept = E // (2 * NSUB)
        base = (c * NSUB + s) * ept
    else:
        ept = E // NSUB
        base = s * ept
    nwin = ept // W

    def load_fire(j, isb, idb, rowsb, gsem):
        off = base + j * W
        pltpu.sync_copy(src_hbm.at[pl.ds(off, W)], isb)
        pltpu.sync_copy(dst_hbm.at[pl.ds(off, W)], idb)
        if not edge_split:
            @pl.loop(0, W, step=16)
            def _(k):
                v = isb[pl.ds(k, 16)]
                isb[pl.ds(k, 16)] = v + v + c
        return pltpu.async_copy(tab.at[isb], rowsb, gsem)

    def scatter(idb, rowsb):
        pltpu.sync_copy(rowsb, acc.at[idb], add=True)

    nwin2 = nwin - (nwin % 2)

    @pl.loop(0, nwin2, step=2)
    def _(j0):
        d0 = load_fire(j0, is0, id0, rows0, gsem0)
        d1 = load_fire(j0 + 1, is1, id1, rows1, gsem1)
        d0.wait()
        scatter(id0, rows0)
        d1.wait()
        scatter(id1, rows1)

    if nwin % 2:
        d0 = load_fire(nwin - 1, is0, id0, rows0, gsem0)
        d0.wait()
        scatter(id0, rows0)

    plsc.subcore_barrier()

    @pl.when(c == 0)
    def _():
        pltpu.sync_copy(acc.at[pl.ds(row0, RPT)], agg_a.at[pl.ds(row0, RPT)])

    @pl.when(c == 1)
    def _():
        pltpu.sync_copy(acc.at[pl.ds(row0, RPT)], agg_b.at[pl.ds(row0, RPT)])


def _sc_cnt_body(dst_hbm, cnt_hbm, id0, hist, red, sbuf, stage):
    c = lax.axis_index("c")
    s = lax.axis_index("s")

    @pl.when(c == 0)
    def _():
        @pl.loop(0, N_ACC, step=16)
        def _(i):
            hist[pl.ds(i, 16)] = jnp.zeros((16,), jnp.float32)

        ept = E // NSUB
        base = s * ept

        @pl.loop(0, ept // W)
        def _(j):
            pltpu.sync_copy(dst_hbm.at[pl.ds(base + j * W, W)], id0)

            @pl.loop(0, W, step=16)
            def _(k):
                plsc.addupdate_scatter(hist, [id0[pl.ds(k, 16)]],
                                       jnp.ones((16,), jnp.float32))

        pltpu.sync_copy(hist, stage.at[s])
        plsc.subcore_barrier()

        chunk0 = s * RPT
        pltpu.sync_copy(stage.at[:, pl.ds(chunk0, RPT)], sbuf)

        @pl.loop(0, RPT, step=16)
        def _(i):
            acc16 = jnp.zeros((16,), jnp.float32)
            for t in range(NSUB):
                acc16 += sbuf[t, pl.ds(i, 16)]
            red[pl.ds(i, 16)] = acc16

        pltpu.sync_copy(red, cnt_hbm.at[pl.ds(chunk0, RPT)])

    @pl.when(c == 1)
    def _():
        plsc.subcore_barrier()


_sc_cnt = pl.kernel(
    _sc_cnt_body,
    out_type=jax.ShapeDtypeStruct((N_ACC,), jnp.float32),
    mesh=_MESH,
    scratch_types=[
        pltpu.VMEM((W,), jnp.int32),
        pltpu.VMEM((N_ACC,), jnp.float32),
        pltpu.VMEM((RPT,), jnp.float32),
        pltpu.VMEM((NSUB, RPT), jnp.float32),
        pltpu.VMEM_SHARED((NSUB, N_ACC), jnp.float32),
    ],
    compiler_params=_SC_CP,
)


def _make_sc_agg(d_half, edge_split):
    out_type = (jax.ShapeDtypeStruct((N_ACC, d_half), jnp.float32),
                jax.ShapeDtypeStruct((N_ACC, d_half), jnp.float32))
    scratch = [
        pltpu.VMEM((W,), jnp.int32),
        pltpu.VMEM((W,), jnp.int32),
        pltpu.VMEM((W,), jnp.int32),
        pltpu.VMEM((W,), jnp.int32),
        pltpu.VMEM((W, d_half), jnp.float32),
        pltpu.VMEM((W, d_half), jnp.float32),
        pltpu.VMEM((ZR, d_half), jnp.float32),
        pltpu.VMEM_SHARED((N_ACC, d_half), jnp.float32),
        pltpu.SemaphoreType.DMA,
        pltpu.SemaphoreType.DMA,
    ]
    return pl.kernel(
        functools.partial(_sc_agg_body, edge_split),
        out_type=out_type,
        mesh=_MESH,
        scratch_types=scratch,
    )


_sc_agg1 = _make_sc_agg(D_IN, True)
_sc_agg2 = _make_sc_agg(D_H // 2, False)


def _dense1_body(x_ref, agga_ref, aggb_ref, cnt_ref, w1lt_ref, b1_ref,
                 w1rt_ref, h1_ref):
    inv = 1.0 / jnp.maximum(cnt_ref[...], 1.0)
    agg = agga_ref[...] + aggb_ref[...]
    mean = agg * inv
    h = jnp.dot(mean, w1lt_ref[...], preferred_element_type=jnp.float32)
    h += jnp.dot(x_ref[...], w1rt_ref[...], preferred_element_type=jnp.float32)
    h += b1_ref[...]
    h1_ref[...] = jnp.maximum(h, 0.0)


def _dense1(x, agg1a, agg1b, cnt_col, w1lt, b1_row, w1rt):
    return pl.pallas_call(
        _dense1_body,
        grid=(NB,),
        in_specs=[
            pl.BlockSpec((R, D_IN), lambda i: (i, 0)),
            pl.BlockSpec((R, D_IN), lambda i: (i, 0)),
            pl.BlockSpec((R, D_IN), lambda i: (i, 0)),
            pl.BlockSpec((R, 1), lambda i: (i, 0)),
            pl.BlockSpec((D_IN, D_H), lambda i: (0, 0)),
            pl.BlockSpec((1, D_H), lambda i: (0, 0)),
            pl.BlockSpec((D_IN, D_H), lambda i: (0, 0)),
        ],
        out_specs=pl.BlockSpec((R, D_H), lambda i: (i, 0)),
        out_shape=jax.ShapeDtypeStruct((N, D_H), jnp.float32),
    )(x, agg1a, agg1b, cnt_col, w1lt, b1_row, w1rt)


def _dense2_body(h1_ref, a2a_ref, a2b_ref, batch_ref, cntr_ref,
                 w2lt_ref, b2_ref, w2rt_ref, out_ref,
                 acc_m2, acc_h1, acc_g):
    j = pl.program_id(0)

    @pl.when(j == 0)
    def _():
        acc_m2[...] = jnp.zeros_like(acc_m2)
        acc_h1[...] = jnp.zeros_like(acc_h1)
        acc_g[...] = jnp.zeros_like(acc_g)

    batch_row = batch_ref[0]
    mask = (jax.lax.broadcasted_iota(jnp.int32, (G, R), 0)
            == batch_row).astype(jnp.float32)
    invr = 1.0 / jnp.maximum(cntr_ref[0], 1.0)
    maskw = mask * invr

    agg2 = jnp.concatenate([a2a_ref[...], a2b_ref[...]], axis=1)
    acc_m2[...] += jnp.dot(maskw, agg2, preferred_element_type=jnp.float32)
    acc_h1[...] += jnp.dot(mask, h1_ref[...],
                           preferred_element_type=jnp.float32)
    acc_g[...] += jnp.sum(mask, axis=1, keepdims=True)

    @pl.when(j == pl.num_programs(0) - 1)
    def _():
        out = jnp.dot(acc_m2[...], w2lt_ref[...],
                      preferred_element_type=jnp.float32)
        out += jnp.dot(acc_h1[...], w2rt_ref[...],
                       preferred_element_type=jnp.float32)
        out += acc_g[...] * b2_ref[...]
        out_ref[...] = out


def _dense2(h1, a2a, a2b, batch3d, cnt3d, w2lt, b2_row, w2rt):
    return pl.pallas_call(
        _dense2_body,
        grid=(NB,),
        in_specs=[
            pl.BlockSpec((R, D_H), lambda i: (i, 0)),
            pl.BlockSpec((R, D_H // 2), lambda i: (i, 0)),
            pl.BlockSpec((R, D_H // 2), lambda i: (i, 0)),
            pl.BlockSpec((1, 1, R), lambda i: (i, 0, 0)),
            pl.BlockSpec((1, 1, R), lambda i: (i, 0, 0)),
            pl.BlockSpec((D_H, D_OUT), lambda i: (0, 0)),
            pl.BlockSpec((1, D_OUT), lambda i: (0, 0)),
            pl.BlockSpec((D_H, D_OUT), lambda i: (0, 0)),
        ],
        out_specs=pl.BlockSpec((G, D_OUT), lambda i: (0, 0)),
        out_shape=jax.ShapeDtypeStruct((G, D_OUT), jnp.float32),
        scratch_shapes=[
            pltpu.VMEM((G, D_OUT), jnp.float32),
            pltpu.VMEM((G, D_OUT), jnp.float32),
            pltpu.VMEM((G, 1), jnp.float32),
        ],
    )(h1, a2a, a2b, batch3d, cnt3d, w2lt, b2_row, w2rt)


@jax.jit
def kernel(x, edge_index, batch, W1_l, b1_l, W1_r, W2_l, b2_l, W2_r):
    src = edge_index[0]
    dst = edge_index[1]

    agg1a, agg1b = _sc_agg1(x, src, dst)
    cntvec = _sc_cnt(dst)

    cnt = cntvec[:N]
    cnt_col = cnt[:, None]
    cnt3d = cnt.reshape(NB, 1, R)

    h1 = _dense1(x, agg1a, agg1b, cnt_col, W1_l.T, b1_l[None, :], W1_r.T)

    htab = h1.reshape(2 * N, D_H // 2)
    agg2a, agg2b = _sc_agg2(htab, src, dst)

    batch3d = batch.reshape(NB, 1, R)

    return _dense2(h1, agg2a, agg2b, batch3d, cnt3d,
                   W2_l.T, b2_l[None, :], W2_r.T)


# speedup vs baseline: 6.3913x; 1.1562x over previous
import dataclasses
import functools

import jax
import jax.numpy as jnp
from jax import lax
from jax.experimental import pallas as pl
from jax.experimental.pallas import tpu as pltpu
from jax.experimental.pallas import tpu_sc as plsc

N = 10000
E = 320000
D_IN = 128
D_H = 256
D_OUT = 256
G = 64

R = 1000
NB = N // R

NSUB = 16
NTILE = 2 * NSUB
W = 80
N_ACC = 10240
RPT = N_ACC // NSUB
ZR = 32

_MESH = plsc.VectorSubcoreMesh(core_axis_name="c", subcore_axis_name="s")

_SC_CP = pltpu.CompilerParams()
if "needs_layout_passes" in pltpu.CompilerParams.__dataclass_fields__:
    _SC_CP = dataclasses.replace(_SC_CP, needs_layout_passes=False)


def _sc_agg_body(edge_split, with_hist, *refs):
    if with_hist:
        (tab, src_hbm, dst_hbm, agg_a, agg_b, hist_hbm,
         ib0, ib1, rows0, rows1, zbuf, hist, acc,
         gsem0, gsem1) = refs
    else:
        (tab, src_hbm, dst_hbm, agg_a, agg_b,
         ib0, ib1, rows0, rows1, zbuf, acc,
         gsem0, gsem1) = refs
        hist = None

    c = lax.axis_index("c")
    s = lax.axis_index("s")
    row0 = s * RPT
    d_half = rows0.shape[1]

    @pl.loop(0, ZR)
    def _(r):
        @pl.loop(0, d_half, step=16)
        def _(cc):
            zbuf[r, pl.ds(cc, 16)] = jnp.zeros((16,), jnp.float32)

    @pl.loop(0, RPT // ZR)
    def _(i):
        pltpu.sync_copy(zbuf, acc.at[pl.ds(row0 + i * ZR, ZR)])

    if with_hist:
        @pl.loop(0, N_ACC, step=16)
        def _(i):
            hist[pl.ds(i, 16)] = jnp.zeros((16,), jnp.float32)

    plsc.subcore_barrier()

    if edge_split:
        ept = E // (2 * NSUB)
        base = (c * NSUB + s) * ept
    else:
        ept = E // NSUB
        base = s * ept
    nwin = ept // W

    def load_fire(j, ib, rowsb, gsem):
        off = base + j * W
        pltpu.sync_copy(src_hbm.at[pl.ds(off, W)], ib.at[0])
        pltpu.sync_copy(dst_hbm.at[pl.ds(off, W)], ib.at[1])
        if not edge_split:
            @pl.loop(0, W, step=16)
            def _(k):
                v = ib[0, pl.ds(k, 16)]
                ib[0, pl.ds(k, 16)] = v + v + c
        return pltpu.async_copy(tab.at[ib.at[0]], rowsb, gsem)

    def scatter(ib, rowsb):
        pltpu.sync_copy(rowsb, acc.at[ib.at[1]], add=True)
        if with_hist:
            @pl.loop(0, W, step=16)
            def _(k):
                plsc.addupdate_scatter(hist, [ib[1, pl.ds(k, 16)]],
                                       jnp.ones((16,), jnp.float32))

    nwin2 = nwin - (nwin % 2)

    @pl.loop(0, nwin2, step=2)
    def _(j0):
        d0 = load_fire(j0, ib0, rows0, gsem0)
        d1 = load_fire(j0 + 1, ib1, rows1, gsem1)
        d0.wait()
        scatter(ib0, rows0)
        d1.wait()
        scatter(ib1, rows1)

    if nwin % 2:
        d0 = load_fire(nwin - 1, ib0, rows0, gsem0)
        d0.wait()
        scatter(ib0, rows0)

    plsc.subcore_barrier()

    @pl.when(c == 0)
    def _():
        pltpu.sync_copy(acc.at[pl.ds(row0, RPT)], agg_a.at[pl.ds(row0, RPT)])

    @pl.when(c == 1)
    def _():
        pltpu.sync_copy(acc.at[pl.ds(row0, RPT)], agg_b.at[pl.ds(row0, RPT)])

    if with_hist:
        pltpu.sync_copy(hist, hist_hbm.at[c * NSUB + s])


def _make_sc_agg(d_half, edge_split, with_hist):
    out_type = [jax.ShapeDtypeStruct((N_ACC, d_half), jnp.float32),
                jax.ShapeDtypeStruct((N_ACC, d_half), jnp.float32)]
    if with_hist:
        out_type.append(jax.ShapeDtypeStruct((NTILE, N_ACC), jnp.float32))
    scratch = [
        pltpu.VMEM((2, W), jnp.int32),
        pltpu.VMEM((2, W), jnp.int32),
        pltpu.VMEM((W, d_half), jnp.float32),
        pltpu.VMEM((W, d_half), jnp.float32),
        pltpu.VMEM((ZR, d_half), jnp.float32),
    ]
    if with_hist:
        scratch.append(pltpu.VMEM((N_ACC,), jnp.float32))
    scratch += [
        pltpu.VMEM_SHARED((N_ACC, d_half), jnp.float32),
        pltpu.SemaphoreType.DMA,
        pltpu.SemaphoreType.DMA,
    ]
    return pl.kernel(
        functools.partial(_sc_agg_body, edge_split, with_hist),
        out_type=tuple(out_type),
        mesh=_MESH,
        scratch_types=scratch,
        compiler_params=_SC_CP,
    )


_sc_agg1 = _make_sc_agg(D_IN, True, True)
_sc_agg2 = _make_sc_agg(D_H // 2, False, False)


def _dense1_body(x_ref, agga_ref, aggb_ref, hist_ref, w1lt_ref, b1_ref,
                 w1rt_ref, h1_ref, cnt_s):
    j = pl.program_id(0)

    @pl.when(j == 0)
    def _():
        cnt_s[...] = lax.dot_general(
            hist_ref[...], jnp.ones((NTILE, 1), jnp.float32),
            (((0,), (0,)), ((), ())),
            preferred_element_type=jnp.float32)

    inv = 1.0 / jnp.maximum(cnt_s[pl.ds(j * R, R), :], 1.0)
    agg = agga_ref[...] + aggb_ref[...]
    mean = agg * inv
    h = jnp.dot(mean, w1lt_ref[...], preferred_element_type=jnp.float32)
    h += jnp.dot(x_ref[...], w1rt_ref[...], preferred_element_type=jnp.float32)
    h += b1_ref[...]
    h1_ref[...] = jnp.maximum(h, 0.0)


def _dense1(x, agg1a, agg1b, hist, w1lt, b1_row, w1rt):
    return pl.pallas_call(
        _dense1_body,
        grid=(NB,),
        in_specs=[
            pl.BlockSpec((R, D_IN), lambda i: (i, 0)),
            pl.BlockSpec((R, D_IN), lambda i: (i, 0)),
            pl.BlockSpec((R, D_IN), lambda i: (i, 0)),
            pl.BlockSpec((NTILE, N_ACC), lambda i: (0, 0)),
            pl.BlockSpec((D_IN, D_H), lambda i: (0, 0)),
            pl.BlockSpec((1, D_H), lambda i: (0, 0)),
            pl.BlockSpec((D_IN, D_H), lambda i: (0, 0)),
        ],
        out_specs=pl.BlockSpec((R, D_H), lambda i: (i, 0)),
        out_shape=jax.ShapeDtypeStruct((N, D_H), jnp.float32),
        scratch_shapes=[pltpu.VMEM((N_ACC, 1), jnp.float32)],
    )(x, agg1a, agg1b, hist, w1lt, b1_row, w1rt)


def _dense2_body(h1_ref, a2a_ref, a2b_ref, batch_ref, hist_ref,
                 w2lt_ref, b2_ref, w2rt_ref, out_ref,
                 acc_m2, acc_h1, acc_g, cnt_s):
    j = pl.program_id(0)

    @pl.when(j == 0)
    def _():
        acc_m2[...] = jnp.zeros_like(acc_m2)
        acc_h1[...] = jnp.zeros_like(acc_h1)
        acc_g[...] = jnp.zeros_like(acc_g)
        cnt_s[...] = lax.dot_general(
            hist_ref[...], jnp.ones((NTILE, 1), jnp.float32),
            (((0,), (0,)), ((), ())),
            preferred_element_type=jnp.float32)

    batch_row = batch_ref[0]
    mask = (jax.lax.broadcasted_iota(jnp.int32, (G, R), 0)
            == batch_row).astype(jnp.float32)
    inv = 1.0 / jnp.maximum(cnt_s[pl.ds(j * R, R), :], 1.0)

    agg2 = jnp.concatenate([a2a_ref[...], a2b_ref[...]], axis=1)
    mean2 = agg2 * inv
    acc_m2[...] += jnp.dot(mask, mean2, preferred_element_type=jnp.float32)
    acc_h1[...] += jnp.dot(mask, h1_ref[...],
                           preferred_element_type=jnp.float32)
    acc_g[...] += jnp.sum(mask, axis=1, keepdims=True)

    @pl.when(j == pl.num_programs(0) - 1)
    def _():
        out = jnp.dot(acc_m2[...], w2lt_ref[...],
                      preferred_element_type=jnp.float32)
        out += jnp.dot(acc_h1[...], w2rt_ref[...],
                       preferred_element_type=jnp.float32)
        out += acc_g[...] * b2_ref[...]
        out_ref[...] = out


def _dense2(h1, a2a, a2b, batch3d, hist, w2lt, b2_row, w2rt):
    return pl.pallas_call(
        _dense2_body,
        grid=(NB,),
        in_specs=[
            pl.BlockSpec((R, D_H), lambda i: (i, 0)),
            pl.BlockSpec((R, D_H // 2), lambda i: (i, 0)),
            pl.BlockSpec((R, D_H // 2), lambda i: (i, 0)),
            pl.BlockSpec((1, 1, R), lambda i: (i, 0, 0)),
            pl.BlockSpec((NTILE, N_ACC), lambda i: (0, 0)),
            pl.BlockSpec((D_H, D_OUT), lambda i: (0, 0)),
            pl.BlockSpec((1, D_OUT), lambda i: (0, 0)),
            pl.BlockSpec((D_H, D_OUT), lambda i: (0, 0)),
        ],
        out_specs=pl.BlockSpec((G, D_OUT), lambda i: (0, 0)),
        out_shape=jax.ShapeDtypeStruct((G, D_OUT), jnp.float32),
        scratch_shapes=[
            pltpu.VMEM((G, D_OUT), jnp.float32),
            pltpu.VMEM((G, D_OUT), jnp.float32),
            pltpu.VMEM((G, 1), jnp.float32),
            pltpu.VMEM((N_ACC, 1), jnp.float32),
        ],
    )(h1, a2a, a2b, batch3d, hist, w2lt, b2_row, w2rt)


@jax.jit
def kernel(x, edge_index, batch, W1_l, b1_l, W1_r, W2_l, b2_l, W2_r):
    src = edge_index[0]
    dst = edge_index[1]
    agg1a, agg1b, hist = _sc_agg1(x, src, dst)

    h1 = _dense1(x, agg1a, agg1b, hist, W1_l.T, b1_l[None, :], W1_r.T)

    htab = h1.reshape(2 * N, D_H // 2)
    agg2a, agg2b = _sc_agg2(htab, src, dst)

    batch3d = batch.reshape(NB, 1, R)

    return _dense2(h1, agg2a, agg2b, batch3d, hist,
                   W2_l.T, b2_l[None, :], W2_r.T)


# speedup vs baseline: 8.7132x; 1.3633x over previous
import dataclasses
import functools

import jax
import jax.numpy as jnp
from jax import lax
from jax.experimental import pallas as pl
from jax.experimental.pallas import tpu as pltpu
from jax.experimental.pallas import tpu_sc as plsc

N = 10000
E = 320000
D_IN = 128
D_H = 256
D_OUT = 256
G = 64

R = 1000
NB = N // R

NSUB = 16
NTILE = 2 * NSUB
W = 80
N_ACC = 10240
RPT = N_ACC // NSUB
ZR = 32

_MESH = plsc.VectorSubcoreMesh(core_axis_name="c", subcore_axis_name="s")

_SC_CP = pltpu.CompilerParams()
if "needs_layout_passes" in pltpu.CompilerParams.__dataclass_fields__:
    _SC_CP = dataclasses.replace(_SC_CP, needs_layout_passes=False)


def _sc_agg_body(edge_split, with_hist, nbuf, *refs):
    if with_hist:
        (tab, src_hbm, dst_hbm, agg_a, agg_b, hist_hbm), rest = \
            refs[:6], refs[6:]
    else:
        (tab, src_hbm, dst_hbm, agg_a, agg_b), rest = refs[:5], refs[5:]
    ibs = rest[:nbuf]
    rows = rest[nbuf:2 * nbuf]
    zbuf = rest[2 * nbuf]
    if with_hist:
        hist = rest[2 * nbuf + 1]
        acc = rest[2 * nbuf + 2]
        sems = rest[2 * nbuf + 3:]
    else:
        hist = None
        acc = rest[2 * nbuf + 1]
        sems = rest[2 * nbuf + 2:]
    isems = sems[:nbuf]
    gsems = sems[nbuf:2 * nbuf]
    ssems = sems[2 * nbuf:3 * nbuf]

    c = lax.axis_index("c")
    s = lax.axis_index("s")
    row0 = s * RPT
    d_half = rows[0].shape[1]

    @pl.loop(0, ZR)
    def _(r):
        @pl.loop(0, d_half, step=16)
        def _(cc):
            zbuf[r, pl.ds(cc, 16)] = jnp.zeros((16,), jnp.float32)

    @pl.loop(0, RPT // ZR)
    def _(i):
        pltpu.sync_copy(zbuf, acc.at[pl.ds(row0 + i * ZR, ZR)])

    if with_hist:
        @pl.loop(0, N_ACC, step=16)
        def _(i):
            hist[pl.ds(i, 16)] = jnp.zeros((16,), jnp.float32)

    plsc.subcore_barrier()

    if edge_split:
        ept = E // (2 * NSUB)
        base = (c * NSUB + s) * ept
    else:
        ept = E // NSUB
        base = s * ept
    nwin = ept // W

    def fire_idx(j, ib, isem):
        off = base + j * W
        da = pltpu.async_copy(src_hbm.at[pl.ds(off, W)], ib.at[0], isem)
        db = pltpu.async_copy(dst_hbm.at[pl.ds(off, W)], ib.at[1], isem)
        return da, db

    def fire_gather(ib, rowsb, gsem):
        if not edge_split:
            @pl.loop(0, W, step=16)
            def _(k):
                v = ib[0, pl.ds(k, 16)]
                ib[0, pl.ds(k, 16)] = v + v + c
        return pltpu.async_copy(tab.at[ib.at[0]], rowsb, gsem)

    def fire_scatter(ib, rowsb, ssem):
        d = pltpu.async_copy(rowsb, acc.at[ib.at[1]], ssem, add=True)
        if with_hist:
            @pl.loop(0, W, step=16)
            def _(k):
                plsc.addupdate_scatter(hist, [ib[1, pl.ds(k, 16)]],
                                       jnp.ones((16,), jnp.float32))
        return d

    nwin4 = nwin - (nwin % nbuf)

    @pl.loop(0, nwin4, step=nbuf)
    def _(j0):
        dis = [fire_idx(j0 + b, ibs[b], isems[b]) for b in range(nbuf)]
        dgs = []
        for b in range(nbuf):
            dis[b][0].wait()
            dis[b][1].wait()
            dgs.append(fire_gather(ibs[b], rows[b], gsems[b]))
        dss = []
        for b in range(nbuf):
            dgs[b].wait()
            dss.append(fire_scatter(ibs[b], rows[b], ssems[b]))
        for b in range(nbuf):
            dss[b].wait()

    if nwin % nbuf:
        @pl.loop(nwin4, nwin)
        def _(j):
            da, db = fire_idx(j, ibs[0], isems[0])
            da.wait()
            db.wait()
            dg = fire_gather(ibs[0], rows[0], gsems[0])
            dg.wait()
            ds = fire_scatter(ibs[0], rows[0], ssems[0])
            ds.wait()

    plsc.subcore_barrier()

    @pl.when(c == 0)
    def _():
        pltpu.sync_copy(acc.at[pl.ds(row0, RPT)], agg_a.at[pl.ds(row0, RPT)])

    @pl.when(c == 1)
    def _():
        pltpu.sync_copy(acc.at[pl.ds(row0, RPT)], agg_b.at[pl.ds(row0, RPT)])

    if with_hist:
        pltpu.sync_copy(hist, hist_hbm.at[c * NSUB + s])


def _make_sc_agg(d_half, edge_split, with_hist, nbuf):
    out_type = [jax.ShapeDtypeStruct((N_ACC, d_half), jnp.float32),
                jax.ShapeDtypeStruct((N_ACC, d_half), jnp.float32)]
    if with_hist:
        out_type.append(jax.ShapeDtypeStruct((NTILE, N_ACC), jnp.float32))
    scratch = [pltpu.VMEM((2, W), jnp.int32) for _ in range(nbuf)]
    scratch += [pltpu.VMEM((W, d_half), jnp.float32) for _ in range(nbuf)]
    scratch.append(pltpu.VMEM((ZR, d_half), jnp.float32))
    if with_hist:
        scratch.append(pltpu.VMEM((N_ACC,), jnp.float32))
    scratch.append(pltpu.VMEM_SHARED((N_ACC, d_half), jnp.float32))
    scratch += [pltpu.SemaphoreType.DMA for _ in range(3 * nbuf)]
    return pl.kernel(
        functools.partial(_sc_agg_body, edge_split, with_hist, nbuf),
        out_type=tuple(out_type),
        mesh=_MESH,
        scratch_types=scratch,
        compiler_params=_SC_CP,
    )


_sc_agg1 = _make_sc_agg(D_IN, True, True, 3)
_sc_agg2 = _make_sc_agg(D_H // 2, False, False, 4)


def _dense1_body(x_ref, agga_ref, aggb_ref, hist_ref, w1lt_ref, b1_ref,
                 w1rt_ref, h1_ref, cnt_s):
    j = pl.program_id(0)

    @pl.when(j == 0)
    def _():
        cnt_s[...] = lax.dot_general(
            hist_ref[...], jnp.ones((NTILE, 1), jnp.float32),
            (((0,), (0,)), ((), ())),
            preferred_element_type=jnp.float32)

    inv = 1.0 / jnp.maximum(cnt_s[pl.ds(j * R, R), :], 1.0)
    agg = agga_ref[...] + aggb_ref[...]
    mean = agg * inv
    h = jnp.dot(mean, w1lt_ref[...], preferred_element_type=jnp.float32)
    h += jnp.dot(x_ref[...], w1rt_ref[...], preferred_element_type=jnp.float32)
    h += b1_ref[...]
    h1_ref[...] = jnp.maximum(h, 0.0)


def _dense1(x, agg1a, agg1b, hist, w1lt, b1_row, w1rt):
    return pl.pallas_call(
        _dense1_body,
        grid=(NB,),
        in_specs=[
            pl.BlockSpec((R, D_IN), lambda i: (i, 0)),
            pl.BlockSpec((R, D_IN), lambda i: (i, 0)),
            pl.BlockSpec((R, D_IN), lambda i: (i, 0)),
            pl.BlockSpec((NTILE, N_ACC), lambda i: (0, 0)),
            pl.BlockSpec((D_IN, D_H), lambda i: (0, 0)),
            pl.BlockSpec((1, D_H), lambda i: (0, 0)),
            pl.BlockSpec((D_IN, D_H), lambda i: (0, 0)),
        ],
        out_specs=pl.BlockSpec((R, D_H), lambda i: (i, 0)),
        out_shape=jax.ShapeDtypeStruct((N, D_H), jnp.float32),
        scratch_shapes=[pltpu.VMEM((N_ACC, 1), jnp.float32)],
    )(x, agg1a, agg1b, hist, w1lt, b1_row, w1rt)


def _dense2_body(h1_ref, a2a_ref, a2b_ref, batch_ref, hist_ref,
                 w2lt_ref, b2_ref, w2rt_ref, out_ref,
                 acc_m2, acc_h1, acc_g, cnt_s):
    j = pl.program_id(0)

    @pl.when(j == 0)
    def _():
        acc_m2[...] = jnp.zeros_like(acc_m2)
        acc_h1[...] = jnp.zeros_like(acc_h1)
        acc_g[...] = jnp.zeros_like(acc_g)
        cnt_s[...] = lax.dot_general(
            hist_ref[...], jnp.ones((NTILE, 1), jnp.float32),
            (((0,), (0,)), ((), ())),
            preferred_element_type=jnp.float32)

    batch_row = batch_ref[0]
    mask = (jax.lax.broadcasted_iota(jnp.int32, (G, R), 0)
            == batch_row).astype(jnp.float32)
    inv = 1.0 / jnp.maximum(cnt_s[pl.ds(j * R, R), :], 1.0)

    agg2 = jnp.concatenate([a2a_ref[...], a2b_ref[...]], axis=1)
    mean2 = agg2 * inv
    acc_m2[...] += jnp.dot(mask, mean2, preferred_element_type=jnp.float32)
    acc_h1[...] += jnp.dot(mask, h1_ref[...],
                           preferred_element_type=jnp.float32)
    acc_g[...] += jnp.sum(mask, axis=1, keepdims=True)

    @pl.when(j == pl.num_programs(0) - 1)
    def _():
        out = jnp.dot(acc_m2[...], w2lt_ref[...],
                      preferred_element_type=jnp.float32)
        out += jnp.dot(acc_h1[...], w2rt_ref[...],
                       preferred_element_type=jnp.float32)
        out += acc_g[...] * b2_ref[...]
        out_ref[...] = out


def _dense2(h1, a2a, a2b, batch3d, hist, w2lt, b2_row, w2rt):
    return pl.pallas_call(
        _dense2_body,
        grid=(NB,),
        in_specs=[
            pl.BlockSpec((R, D_H), lambda i: (i, 0)),
            pl.BlockSpec((R, D_H // 2), lambda i: (i, 0)),
            pl.BlockSpec((R, D_H // 2), lambda i: (i, 0)),
            pl.BlockSpec((1, 1, R), lambda i: (i, 0, 0)),
            pl.BlockSpec((NTILE, N_ACC), lambda i: (0, 0)),
            pl.BlockSpec((D_H, D_OUT), lambda i: (0, 0)),
            pl.BlockSpec((1, D_OUT), lambda i: (0, 0)),
            pl.BlockSpec((D_H, D_OUT), lambda i: (0, 0)),
        ],
        out_specs=pl.BlockSpec((G, D_OUT), lambda i: (0, 0)),
        out_shape=jax.ShapeDtypeStruct((G, D_OUT), jnp.float32),
        scratch_shapes=[
            pltpu.VMEM((G, D_OUT), jnp.float32),
            pltpu.VMEM((G, D_OUT), jnp.float32),
            pltpu.VMEM((G, 1), jnp.float32),
            pltpu.VMEM((N_ACC, 1), jnp.float32),
        ],
    )(h1, a2a, a2b, batch3d, hist, w2lt, b2_row, w2rt)


@jax.jit
def kernel(x, edge_index, batch, W1_l, b1_l, W1_r, W2_l, b2_l, W2_r):
    src = edge_index[0]
    dst = edge_index[1]
    agg1a, agg1b, hist = _sc_agg1(x, src, dst)

    h1 = _dense1(x, agg1a, agg1b, hist, W1_l.T, b1_l[None, :], W1_r.T)

    htab = h1.reshape(2 * N, D_H // 2)
    agg2a, agg2b = _sc_agg2(htab, src, dst)

    batch3d = batch.reshape(NB, 1, R)

    return _dense2(h1, agg2a, agg2b, batch3d, hist,
                   W2_l.T, b2_l[None, :], W2_r.T)


# speedup vs baseline: 8.9179x; 1.0235x over previous
import dataclasses
import functools

import jax
import jax.numpy as jnp
from jax import lax
from jax.experimental import pallas as pl
from jax.experimental.pallas import tpu as pltpu
from jax.experimental.pallas import tpu_sc as plsc

N = 10000
E = 320000
D_IN = 128
D_H = 256
D_OUT = 256
G = 64

R = 1000
NB = N // R

NSUB = 16
NTILE = 2 * NSUB
N_ACC = 10240
RPT = N_ACC // NSUB
ZR = 16

_MESH = plsc.VectorSubcoreMesh(core_axis_name="c", subcore_axis_name="s")

_SC_CP = pltpu.CompilerParams()
if "needs_layout_passes" in pltpu.CompilerParams.__dataclass_fields__:
    _SC_CP = dataclasses.replace(_SC_CP, needs_layout_passes=False)


def _sc_agg_body(edge_split, with_hist, nbuf, W, *refs):
    if with_hist:
        (tab, src_hbm, dst_hbm, agg_a, agg_b, hist_hbm), rest = \
            refs[:6], refs[6:]
    else:
        (tab, src_hbm, dst_hbm, agg_a, agg_b), rest = refs[:5], refs[5:]
    ibs = rest[:nbuf]
    rows = rest[nbuf:2 * nbuf]
    zbuf = rest[2 * nbuf]
    if with_hist:
        hist = rest[2 * nbuf + 1]
        acc = rest[2 * nbuf + 2]
        sems = rest[2 * nbuf + 3:]
    else:
        hist = None
        acc = rest[2 * nbuf + 1]
        sems = rest[2 * nbuf + 2:]
    isems = sems[:nbuf]
    gsems = sems[nbuf:2 * nbuf]
    ssems = sems[2 * nbuf:3 * nbuf]
    if len(sems) > 3 * nbuf:
        ibt, rowst = sems[3 * nbuf:]
    else:
        ibt = rowst = None

    c = lax.axis_index("c")
    s = lax.axis_index("s")
    row0 = s * RPT
    d_half = rows[0].shape[1]

    @pl.loop(0, ZR)
    def _(r):
        @pl.loop(0, d_half, step=16)
        def _(cc):
            zbuf[r, pl.ds(cc, 16)] = jnp.zeros((16,), jnp.float32)

    @pl.loop(0, RPT // ZR)
    def _(i):
        pltpu.sync_copy(zbuf, acc.at[pl.ds(row0 + i * ZR, ZR)])

    if with_hist:
        @pl.loop(0, N_ACC, step=16)
        def _(i):
            hist[pl.ds(i, 16)] = jnp.zeros((16,), jnp.float32)

    plsc.subcore_barrier()

    if edge_split:
        ept = E // (2 * NSUB)
        base = (c * NSUB + s) * ept
    else:
        ept = E // NSUB
        base = s * ept
    nwin = ept // W
    tail = ept - nwin * W

    def fire_idx(j, ib, isem):
        off = base + j * W
        da = pltpu.async_copy(src_hbm.at[pl.ds(off, W)], ib.at[0], isem)
        db = pltpu.async_copy(dst_hbm.at[pl.ds(off, W)], ib.at[1], isem)
        return da, db

    def fire_gather(ib, rowsb, gsem):
        if not edge_split:
            @pl.loop(0, W, step=16)
            def _(k):
                v = ib[0, pl.ds(k, 16)]
                ib[0, pl.ds(k, 16)] = v + v + c
        return pltpu.async_copy(tab.at[ib.at[0]], rowsb, gsem)

    def fire_scatter(ib, rowsb, ssem):
        d = pltpu.async_copy(rowsb, acc.at[ib.at[1]], ssem, add=True)
        if with_hist:
            @pl.loop(0, W, step=16)
            def _(k):
                plsc.addupdate_scatter(hist, [ib[1, pl.ds(k, 16)]],
                                       jnp.ones((16,), jnp.float32))
        return d

    nwin4 = nwin - (nwin % nbuf)

    @pl.loop(0, nwin4, step=nbuf)
    def _(j0):
        dis = [fire_idx(j0 + b, ibs[b], isems[b]) for b in range(nbuf)]
        dgs = []
        for b in range(nbuf):
            dis[b][0].wait()
            dis[b][1].wait()
            dgs.append(fire_gather(ibs[b], rows[b], gsems[b]))
        dss = []
        for b in range(nbuf):
            dgs[b].wait()
            dss.append(fire_scatter(ibs[b], rows[b], ssems[b]))
        for b in range(nbuf):
            dss[b].wait()

    if nwin % nbuf:
        @pl.loop(nwin4, nwin)
        def _(j):
            da, db = fire_idx(j, ibs[0], isems[0])
            da.wait()
            db.wait()
            dg = fire_gather(ibs[0], rows[0], gsems[0])
            dg.wait()
            ds = fire_scatter(ibs[0], rows[0], ssems[0])
            ds.wait()

    if tail:
        toff = base + nwin * W
        da = pltpu.async_copy(src_hbm.at[pl.ds(toff, tail)], ibt.at[0],
                              isems[0])
        db = pltpu.async_copy(dst_hbm.at[pl.ds(toff, tail)], ibt.at[1],
                              isems[0])
        da.wait()
        db.wait()
        if not edge_split:
            @pl.loop(0, tail, step=16)
            def _(k):
                v = ibt[0, pl.ds(k, 16)]
                ibt[0, pl.ds(k, 16)] = v + v + c
        dg = pltpu.async_copy(tab.at[ibt.at[0]], rowst, gsems[0])
        dg.wait()
        ds = pltpu.async_copy(rowst, acc.at[ibt.at[1]], ssems[0], add=True)
        if with_hist:
            @pl.loop(0, tail, step=16)
            def _(k):
                plsc.addupdate_scatter(hist, [ibt[1, pl.ds(k, 16)]],
                                       jnp.ones((16,), jnp.float32))
        ds.wait()

    plsc.subcore_barrier()

    @pl.when(c == 0)
    def _():
        pltpu.sync_copy(acc.at[pl.ds(row0, RPT)], agg_a.at[pl.ds(row0, RPT)])

    @pl.when(c == 1)
    def _():
        pltpu.sync_copy(acc.at[pl.ds(row0, RPT)], agg_b.at[pl.ds(row0, RPT)])

    if with_hist:
        pltpu.sync_copy(hist, hist_hbm.at[c * NSUB + s])


def _make_sc_agg(d_half, edge_split, with_hist, nbuf, W):
    out_type = [jax.ShapeDtypeStruct((N_ACC, d_half), jnp.float32),
                jax.ShapeDtypeStruct((N_ACC, d_half), jnp.float32)]
    if with_hist:
        out_type.append(jax.ShapeDtypeStruct((NTILE, N_ACC), jnp.float32))
    scratch = [pltpu.VMEM((2, W), jnp.int32) for _ in range(nbuf)]
    scratch += [pltpu.VMEM((W, d_half), jnp.float32) for _ in range(nbuf)]
    scratch.append(pltpu.VMEM((ZR, d_half), jnp.float32))
    if with_hist:
        scratch.append(pltpu.VMEM((N_ACC,), jnp.float32))
    scratch.append(pltpu.VMEM_SHARED((N_ACC, d_half), jnp.float32))
    scratch += [pltpu.SemaphoreType.DMA for _ in range(3 * nbuf)]
    ept = (E // (2 * NSUB)) if edge_split else (E // NSUB)
    tail = ept % W
    if tail:
        scratch += [pltpu.VMEM((2, tail), jnp.int32),
                    pltpu.VMEM((tail, d_half), jnp.float32)]
    return pl.kernel(
        functools.partial(_sc_agg_body, edge_split, with_hist, nbuf, W),
        out_type=tuple(out_type),
        mesh=_MESH,
        scratch_types=scratch,
        compiler_params=_SC_CP,
    )


_sc_agg1 = _make_sc_agg(D_IN, True, True, 4, 64)
_sc_agg2 = _make_sc_agg(D_H // 2, False, False, 5, 64)


def _dense1_body(x_ref, agga_ref, aggb_ref, hist_ref, w1lt_ref, b1_ref,
                 w1rt_ref, h1_ref, cnt_s):
    j = pl.program_id(0)

    @pl.when(j == 0)
    def _():
        cnt_s[...] = lax.dot_general(
            hist_ref[...], jnp.ones((NTILE, 1), jnp.float32),
            (((0,), (0,)), ((), ())),
            preferred_element_type=jnp.float32)

    inv = 1.0 / jnp.maximum(cnt_s[pl.ds(j * R, R), :], 1.0)
    agg = agga_ref[...] + aggb_ref[...]
    mean = agg * inv
    h = jnp.dot(mean, w1lt_ref[...], preferred_element_type=jnp.float32)
    h += jnp.dot(x_ref[...], w1rt_ref[...], preferred_element_type=jnp.float32)
    h += b1_ref[...]
    h1_ref[...] = jnp.maximum(h, 0.0)


def _dense1(x, agg1a, agg1b, hist, w1lt, b1_row, w1rt):
    return pl.pallas_call(
        _dense1_body,
        grid=(NB,),
        in_specs=[
            pl.BlockSpec((R, D_IN), lambda i: (i, 0)),
            pl.BlockSpec((R, D_IN), lambda i: (i, 0)),
            pl.BlockSpec((R, D_IN), lambda i: (i, 0)),
            pl.BlockSpec((NTILE, N_ACC), lambda i: (0, 0)),
            pl.BlockSpec((D_IN, D_H), lambda i: (0, 0)),
            pl.BlockSpec((1, D_H), lambda i: (0, 0)),
            pl.BlockSpec((D_IN, D_H), lambda i: (0, 0)),
        ],
        out_specs=pl.BlockSpec((R, D_H), lambda i: (i, 0)),
        out_shape=jax.ShapeDtypeStruct((N, D_H), jnp.float32),
        scratch_shapes=[pltpu.VMEM((N_ACC, 1), jnp.float32)],
    )(x, agg1a, agg1b, hist, w1lt, b1_row, w1rt)


def _dense2_body(h1_ref, a2a_ref, a2b_ref, batch_ref, hist_ref,
                 w2lt_ref, b2_ref, w2rt_ref, out_ref,
                 acc_m2, acc_h1, acc_g, cnt_s):
    j = pl.program_id(0)

    @pl.when(j == 0)
    def _():
        acc_m2[...] = jnp.zeros_like(acc_m2)
        acc_h1[...] = jnp.zeros_like(acc_h1)
        acc_g[...] = jnp.zeros_like(acc_g)
        cnt_s[...] = lax.dot_general(
            hist_ref[...], jnp.ones((NTILE, 1), jnp.float32),
            (((0,), (0,)), ((), ())),
            preferred_element_type=jnp.float32)

    batch_row = batch_ref[0]
    mask = (jax.lax.broadcasted_iota(jnp.int32, (G, R), 0)
            == batch_row).astype(jnp.float32)
    inv = 1.0 / jnp.maximum(cnt_s[pl.ds(j * R, R), :], 1.0)

    agg2 = jnp.concatenate([a2a_ref[...], a2b_ref[...]], axis=1)
    mean2 = agg2 * inv
    acc_m2[...] += jnp.dot(mask, mean2, preferred_element_type=jnp.float32)
    acc_h1[...] += jnp.dot(mask, h1_ref[...],
                           preferred_element_type=jnp.float32)
    acc_g[...] += jnp.sum(mask, axis=1, keepdims=True)

    @pl.when(j == pl.num_programs(0) - 1)
    def _():
        out = jnp.dot(acc_m2[...], w2lt_ref[...],
                      preferred_element_type=jnp.float32)
        out += jnp.dot(acc_h1[...], w2rt_ref[...],
                       preferred_element_type=jnp.float32)
        out += acc_g[...] * b2_ref[...]
        out_ref[...] = out


def _dense2(h1, a2a, a2b, batch3d, hist, w2lt, b2_row, w2rt):
    return pl.pallas_call(
        _dense2_body,
        grid=(NB,),
        in_specs=[
            pl.BlockSpec((R, D_H), lambda i: (i, 0)),
            pl.BlockSpec((R, D_H // 2), lambda i: (i, 0)),
            pl.BlockSpec((R, D_H // 2), lambda i: (i, 0)),
            pl.BlockSpec((1, 1, R), lambda i: (i, 0, 0)),
            pl.BlockSpec((NTILE, N_ACC), lambda i: (0, 0)),
            pl.BlockSpec((D_H, D_OUT), lambda i: (0, 0)),
            pl.BlockSpec((1, D_OUT), lambda i: (0, 0)),
            pl.BlockSpec((D_H, D_OUT), lambda i: (0, 0)),
        ],
        out_specs=pl.BlockSpec((G, D_OUT), lambda i: (0, 0)),
        out_shape=jax.ShapeDtypeStruct((G, D_OUT), jnp.float32),
        scratch_shapes=[
            pltpu.VMEM((G, D_OUT), jnp.float32),
            pltpu.VMEM((G, D_OUT), jnp.float32),
            pltpu.VMEM((G, 1), jnp.float32),
            pltpu.VMEM((N_ACC, 1), jnp.float32),
        ],
    )(h1, a2a, a2b, batch3d, hist, w2lt, b2_row, w2rt)


@jax.jit
def kernel(x, edge_index, batch, W1_l, b1_l, W1_r, W2_l, b2_l, W2_r):
    src = edge_index[0]
    dst = edge_index[1]
    agg1a, agg1b, hist = _sc_agg1(x, src, dst)

    h1 = _dense1(x, agg1a, agg1b, hist, W1_l.T, b1_l[None, :], W1_r.T)

    htab = h1.reshape(2 * N, D_H // 2)
    agg2a, agg2b = _sc_agg2(htab, src, dst)

    batch3d = batch.reshape(NB, 1, R)

    return _dense2(h1, agg2a, agg2b, batch3d, hist,
                   W2_l.T, b2_l[None, :], W2_r.T)


# speedup vs baseline: 8.9316x; 1.0015x over previous
import dataclasses
import functools

import jax
import jax.numpy as jnp
from jax import lax
from jax.experimental import pallas as pl
from jax.experimental.pallas import tpu as pltpu
from jax.experimental.pallas import tpu_sc as plsc

N = 10000
E = 320000
D_IN = 128
D_H = 256
D_OUT = 256
G = 64

R = 1000
NB = N // R

NSUB = 16
NTILE = 2 * NSUB
N_ACC = 10240
RPT = N_ACC // NSUB
ZR = 16

_MESH = plsc.VectorSubcoreMesh(core_axis_name="c", subcore_axis_name="s")

_SC_CP = pltpu.CompilerParams()
if "needs_layout_passes" in pltpu.CompilerParams.__dataclass_fields__:
    _SC_CP = dataclasses.replace(_SC_CP, needs_layout_passes=False)


def _sc_agg_body(edge_split, with_hist, nbuf, W, *refs):
    if with_hist:
        (tab, src_hbm, dst_hbm, agg_a, agg_b, hist_hbm), rest = \
            refs[:6], refs[6:]
    else:
        (tab, src_hbm, dst_hbm, agg_a, agg_b), rest = refs[:5], refs[5:]
    ibs = rest[:nbuf]
    rows = rest[nbuf:2 * nbuf]
    zbuf = rest[2 * nbuf]
    if with_hist:
        hist = rest[2 * nbuf + 1]
        acc = rest[2 * nbuf + 2]
        sems = rest[2 * nbuf + 3:]
    else:
        hist = None
        acc = rest[2 * nbuf + 1]
        sems = rest[2 * nbuf + 2:]
    isems = sems[:nbuf]
    gsems = sems[nbuf:2 * nbuf]
    ssems = sems[2 * nbuf:3 * nbuf]
    if len(sems) > 3 * nbuf:
        ibt, rowst = sems[3 * nbuf:]
    else:
        ibt = rowst = None

    c = lax.axis_index("c")
    s = lax.axis_index("s")
    row0 = s * RPT
    d_half = rows[0].shape[1]

    @pl.loop(0, ZR)
    def _(r):
        @pl.loop(0, d_half, step=16)
        def _(cc):
            zbuf[r, pl.ds(cc, 16)] = jnp.zeros((16,), jnp.float32)

    @pl.loop(0, RPT // ZR)
    def _(i):
        pltpu.sync_copy(zbuf, acc.at[pl.ds(row0 + i * ZR, ZR)])

    if with_hist:
        @pl.loop(0, N_ACC, step=16)
        def _(i):
            hist[pl.ds(i, 16)] = jnp.zeros((16,), jnp.float32)

    plsc.subcore_barrier()

    if edge_split:
        ept = E // (2 * NSUB)
        base = (c * NSUB + s) * ept
    else:
        ept = E // NSUB
        base = s * ept
    nwin = ept // W
    tail = ept - nwin * W

    def fire_idx(j, ib, isem):
        off = base + j * W
        da = pltpu.async_copy(src_hbm.at[pl.ds(off, W)], ib.at[0], isem)
        db = pltpu.async_copy(dst_hbm.at[pl.ds(off, W)], ib.at[1], isem)
        return da, db

    def fire_gather(ib, rowsb, gsem):
        if not edge_split:
            @pl.loop(0, W, step=16)
            def _(k):
                v = ib[0, pl.ds(k, 16)]
                ib[0, pl.ds(k, 16)] = v + v + c
        return pltpu.async_copy(tab.at[ib.at[0]], rowsb, gsem)

    def fire_scatter(ib, rowsb, ssem):
        d = pltpu.async_copy(rowsb, acc.at[ib.at[1]], ssem, add=True)
        if with_hist:
            @pl.loop(0, W, step=16)
            def _(k):
                plsc.addupdate_scatter(hist, [ib[1, pl.ds(k, 16)]],
                                       jnp.ones((16,), jnp.float32))
        return d

    nwin4 = nwin - (nwin % nbuf)

    @pl.loop(0, nwin4, step=nbuf)
    def _(j0):
        dis = [fire_idx(j0 + b, ibs[b], isems[b]) for b in range(nbuf)]
        dgs = []
        for b in range(nbuf):
            dis[b][0].wait()
            dis[b][1].wait()
            dgs.append(fire_gather(ibs[b], rows[b], gsems[b]))
        dss = []
        for b in range(nbuf):
            dgs[b].wait()
            dss.append(fire_scatter(ibs[b], rows[b], ssems[b]))
        for b in range(nbuf):
            dss[b].wait()

    if nwin % nbuf:
        @pl.loop(nwin4, nwin)
        def _(j):
            da, db = fire_idx(j, ibs[0], isems[0])
            da.wait()
            db.wait()
            dg = fire_gather(ibs[0], rows[0], gsems[0])
            dg.wait()
            ds = fire_scatter(ibs[0], rows[0], ssems[0])
            ds.wait()

    if tail:
        toff = base + nwin * W
        da = pltpu.async_copy(src_hbm.at[pl.ds(toff, tail)], ibt.at[0],
                              isems[0])
        db = pltpu.async_copy(dst_hbm.at[pl.ds(toff, tail)], ibt.at[1],
                              isems[0])
        da.wait()
        db.wait()
        if not edge_split:
            @pl.loop(0, tail, step=16)
            def _(k):
                v = ibt[0, pl.ds(k, 16)]
                ibt[0, pl.ds(k, 16)] = v + v + c
        dg = pltpu.async_copy(tab.at[ibt.at[0]], rowst, gsems[0])
        dg.wait()
        ds = pltpu.async_copy(rowst, acc.at[ibt.at[1]], ssems[0], add=True)
        if with_hist:
            @pl.loop(0, tail, step=16)
            def _(k):
                plsc.addupdate_scatter(hist, [ibt[1, pl.ds(k, 16)]],
                                       jnp.ones((16,), jnp.float32))
        ds.wait()

    plsc.subcore_barrier()

    @pl.when(c == 0)
    def _():
        pltpu.sync_copy(acc.at[pl.ds(row0, RPT)], agg_a.at[pl.ds(row0, RPT)])

    @pl.when(c == 1)
    def _():
        pltpu.sync_copy(acc.at[pl.ds(row0, RPT)], agg_b.at[pl.ds(row0, RPT)])

    if with_hist:
        pltpu.sync_copy(hist, hist_hbm.at[c * NSUB + s])


def _make_sc_agg(d_half, edge_split, with_hist, nbuf, W):
    out_type = [jax.ShapeDtypeStruct((N_ACC, d_half), jnp.float32),
                jax.ShapeDtypeStruct((N_ACC, d_half), jnp.float32)]
    if with_hist:
        out_type.append(jax.ShapeDtypeStruct((NTILE, N_ACC), jnp.float32))
    scratch = [pltpu.VMEM((2, W), jnp.int32) for _ in range(nbuf)]
    scratch += [pltpu.VMEM((W, d_half), jnp.float32) for _ in range(nbuf)]
    scratch.append(pltpu.VMEM((ZR, d_half), jnp.float32))
    if with_hist:
        scratch.append(pltpu.VMEM((N_ACC,), jnp.float32))
    scratch.append(pltpu.VMEM_SHARED((N_ACC, d_half), jnp.float32))
    scratch += [pltpu.SemaphoreType.DMA for _ in range(3 * nbuf)]
    ept = (E // (2 * NSUB)) if edge_split else (E // NSUB)
    tail = ept % W
    if tail:
        scratch += [pltpu.VMEM((2, tail), jnp.int32),
                    pltpu.VMEM((tail, d_half), jnp.float32)]
    return pl.kernel(
        functools.partial(_sc_agg_body, edge_split, with_hist, nbuf, W),
        out_type=tuple(out_type),
        mesh=_MESH,
        scratch_types=scratch,
        compiler_params=_SC_CP,
    )


_sc_agg1 = _make_sc_agg(D_IN, True, True, 4, 64)
_sc_agg2 = _make_sc_agg(D_H // 2, False, False, 5, 64)


def _xr_body(x_ref, w1rt_ref, b1_ref, xr_ref):
    xr_ref[...] = (jnp.dot(x_ref[...], w1rt_ref[...],
                           preferred_element_type=jnp.float32)
                   + b1_ref[...])


def _xr(x, w1rt, b1_row):
    return pl.pallas_call(
        _xr_body,
        grid=(NB,),
        in_specs=[
            pl.BlockSpec((R, D_IN), lambda i: (i, 0)),
            pl.BlockSpec((D_IN, D_H), lambda i: (0, 0)),
            pl.BlockSpec((1, D_H), lambda i: (0, 0)),
        ],
        out_specs=pl.BlockSpec((R, D_H), lambda i: (i, 0)),
        out_shape=jax.ShapeDtypeStruct((N, D_H), jnp.float32),
    )(x, w1rt, b1_row)


def _poolh1_body(h1_ref, batch_ref, out_ref, acc):
    j = pl.program_id(0)

    @pl.when(j == 0)
    def _():
        acc[...] = jnp.zeros_like(acc)

    batch_row = batch_ref[0]
    mask = (jax.lax.broadcasted_iota(jnp.int32, (G, R), 0)
            == batch_row).astype(jnp.float32)
    acc[...] += jnp.dot(mask, h1_ref[...],
                        preferred_element_type=jnp.float32)

    @pl.when(j == pl.num_programs(0) - 1)
    def _():
        out_ref[...] = acc[...]


def _poolh1(h1, batch3d):
    return pl.pallas_call(
        _poolh1_body,
        grid=(NB,),
        in_specs=[
            pl.BlockSpec((R, D_H), lambda i: (i, 0)),
            pl.BlockSpec((1, 1, R), lambda i: (i, 0, 0)),
        ],
        out_specs=pl.BlockSpec((G, D_H), lambda i: (0, 0)),
        out_shape=jax.ShapeDtypeStruct((G, D_H), jnp.float32),
        scratch_shapes=[pltpu.VMEM((G, D_H), jnp.float32)],
    )(h1, batch3d)


def _dense1_body(xr_ref, agga_ref, aggb_ref, hist_ref, w1lt_ref,
                 h1_ref, cnt_s):
    j = pl.program_id(0)

    @pl.when(j == 0)
    def _():
        cnt_s[...] = lax.dot_general(
            hist_ref[...], jnp.ones((NTILE, 1), jnp.float32),
            (((0,), (0,)), ((), ())),
            preferred_element_type=jnp.float32)

    inv = 1.0 / jnp.maximum(cnt_s[pl.ds(j * R, R), :], 1.0)
    agg = agga_ref[...] + aggb_ref[...]
    mean = agg * inv
    h = jnp.dot(mean, w1lt_ref[...], preferred_element_type=jnp.float32)
    h += xr_ref[...]
    h1_ref[...] = jnp.maximum(h, 0.0)


def _dense1(xr, agg1a, agg1b, hist, w1lt):
    return pl.pallas_call(
        _dense1_body,
        grid=(NB,),
        in_specs=[
            pl.BlockSpec((R, D_H), lambda i: (i, 0)),
            pl.BlockSpec((R, D_IN), lambda i: (i, 0)),
            pl.BlockSpec((R, D_IN), lambda i: (i, 0)),
            pl.BlockSpec((NTILE, N_ACC), lambda i: (0, 0)),
            pl.BlockSpec((D_IN, D_H), lambda i: (0, 0)),
        ],
        out_specs=pl.BlockSpec((R, D_H), lambda i: (i, 0)),
        out_shape=jax.ShapeDtypeStruct((N, D_H), jnp.float32),
        scratch_shapes=[pltpu.VMEM((N_ACC, 1), jnp.float32)],
    )(xr, agg1a, agg1b, hist, w1lt)


def _dense2_body(a2a_ref, a2b_ref, batch_ref, hist_ref, ph1_ref,
                 w2lt_ref, b2_ref, w2rt_ref, out_ref,
                 acc_m2, acc_g, cnt_s):
    j = pl.program_id(0)

    @pl.when(j == 0)
    def _():
        acc_m2[...] = jnp.zeros_like(acc_m2)
        acc_g[...] = jnp.zeros_like(acc_g)
        cnt_s[...] = lax.dot_general(
            hist_ref[...], jnp.ones((NTILE, 1), jnp.float32),
            (((0,), (0,)), ((), ())),
            preferred_element_type=jnp.float32)

    batch_row = batch_ref[0]
    mask = (jax.lax.broadcasted_iota(jnp.int32, (G, R), 0)
            == batch_row).astype(jnp.float32)
    inv = 1.0 / jnp.maximum(cnt_s[pl.ds(j * R, R), :], 1.0)

    agg2 = jnp.concatenate([a2a_ref[...], a2b_ref[...]], axis=1)
    mean2 = agg2 * inv
    acc_m2[...] += jnp.dot(mask, mean2, preferred_element_type=jnp.float32)
    acc_g[...] += jnp.sum(mask, axis=1, keepdims=True)

    @pl.when(j == pl.num_programs(0) - 1)
    def _():
        out = jnp.dot(acc_m2[...], w2lt_ref[...],
                      preferred_element_type=jnp.float32)
        out += jnp.dot(ph1_ref[...], w2rt_ref[...],
                       preferred_element_type=jnp.float32)
        out += acc_g[...] * b2_ref[...]
        out_ref[...] = out


def _dense2(a2a, a2b, batch3d, hist, ph1, w2lt, b2_row, w2rt):
    return pl.pallas_call(
        _dense2_body,
        grid=(NB,),
        in_specs=[
            pl.BlockSpec((R, D_H // 2), lambda i: (i, 0)),
            pl.BlockSpec((R, D_H // 2), lambda i: (i, 0)),
            pl.BlockSpec((1, 1, R), lambda i: (i, 0, 0)),
            pl.BlockSpec((NTILE, N_ACC), lambda i: (0, 0)),
            pl.BlockSpec((G, D_H), lambda i: (0, 0)),
            pl.BlockSpec((D_H, D_OUT), lambda i: (0, 0)),
            pl.BlockSpec((1, D_OUT), lambda i: (0, 0)),
            pl.BlockSpec((D_H, D_OUT), lambda i: (0, 0)),
        ],
        out_specs=pl.BlockSpec((G, D_OUT), lambda i: (0, 0)),
        out_shape=jax.ShapeDtypeStruct((G, D_OUT), jnp.float32),
        scratch_shapes=[
            pltpu.VMEM((G, D_OUT), jnp.float32),
            pltpu.VMEM((G, 1), jnp.float32),
            pltpu.VMEM((N_ACC, 1), jnp.float32),
        ],
    )(a2a, a2b, batch3d, hist, ph1, w2lt, b2_row, w2rt)


@jax.jit
def kernel(x, edge_index, batch, W1_l, b1_l, W1_r, W2_l, b2_l, W2_r):
    src = edge_index[0]
    dst = edge_index[1]
    agg1a, agg1b, hist = _sc_agg1(x, src, dst)
    xr = _xr(x, W1_r.T, b1_l[None, :])

    h1 = _dense1(xr, agg1a, agg1b, hist, W1_l.T)

    htab = h1.reshape(2 * N, D_H // 2)
    agg2a, agg2b = _sc_agg2(htab, src, dst)

    batch3d = batch.reshape(NB, 1, R)
    ph1 = _poolh1(h1, batch3d)

    return _dense2(agg2a, agg2b, batch3d, hist, ph1,
                   W2_l.T, b2_l[None, :], W2_r.T)
